# baseline (speedup 1.0000x reference)
import functools

import jax
import jax.numpy as jnp
from jax import lax
from jax.experimental import pallas as pl
from jax.experimental.pallas import tpu as pltpu
from jax.experimental.pallas import tpu_sc as plsc

E = 8
TOPK = 2
H = 1024
I = 768
T = 2048
A = T * TOPK
BM = 128
NB = 40
SPAD = NB * BM
LANES = 128

NTILES = 16
NWORK = 2 * NTILES
CHUNK = A // NTILES
GROWS = SPAD // NWORK
ZROWS = SPAD // NTILES


def _router_body(x_ref, gw_ref, logits_ref, ei_ref, w_ref):
    x = x_ref[...]
    logits = jnp.dot(x, gw_ref[...], preferred_element_type=jnp.float32)
    lane = lax.broadcasted_iota(jnp.int32, logits.shape, 1)
    valid = lane < E
    neg = jnp.float32(-1e30)
    ml = jnp.where(valid, logits, neg)
    mx = jnp.max(ml, axis=1, keepdims=True)
    ex = jnp.where(valid, jnp.exp(ml - mx), 0.0)
    p = ex / jnp.sum(ex, axis=1, keepdims=True)
    m1 = jnp.max(p, axis=1, keepdims=True)
    e1 = jnp.min(jnp.where(p == m1, lane, 9999), axis=1, keepdims=True)
    p2 = jnp.where(lane == e1, -1.0, p)
    m2 = jnp.max(p2, axis=1, keepdims=True)
    e2 = jnp.min(jnp.where(p2 == m2, lane, 9999), axis=1, keepdims=True)
    tot = m1 + m2
    logits_ref[...] = logits
    ei_ref[...] = jnp.where(lane == 0, e1, jnp.where(lane == 1, e2, 0)).astype(jnp.int32)
    w_ref[...] = jnp.where(lane == 0, m1 / tot, jnp.where(lane == 1, m2 / tot, 0.0))


def _router(x, gw_pad):
    rb = 256
    return pl.pallas_call(
        _router_body,
        grid=(T // rb,),
        in_specs=[
            pl.BlockSpec((rb, H), lambda i: (i, 0)),
            pl.BlockSpec((H, LANES), lambda i: (0, 0)),
        ],
        out_specs=[
            pl.BlockSpec((rb, LANES), lambda i: (i, 0)),
            pl.BlockSpec((rb, LANES), lambda i: (i, 0)),
            pl.BlockSpec((rb, LANES), lambda i: (i, 0)),
        ],
        out_shape=[
            jax.ShapeDtypeStruct((T, LANES), jnp.float32),
            jax.ShapeDtypeStruct((T, LANES), jnp.int32),
            jax.ShapeDtypeStruct((T, LANES), jnp.float32),
        ],
    )(x, gw_pad)


def _route_body(e_hbm, x_hbm, xs_hbm, be_hbm, pos_hbm,
                eid_v, tmp16, base16, pos_v, tidbuf, pidx, be_v, gidx_v,
                rowsbuf, cnt_sh, row_sh, sem):
    sid = lax.axis_index("s")
    gwid = sid
    abase = sid * CHUNK
    lane16 = lax.iota(jnp.int32, 16)
    zeros16i = jnp.zeros((16,), jnp.int32)

    pltpu.sync_copy(e_hbm.at[pl.ds(abase, CHUNK)], eid_v)

    def _zero(i, c):
        gidx_v[pl.ds(i * 16, 16)] = zeros16i
        return c
    lax.fori_loop(0, ZROWS // 16, _zero, 0)
    pltpu.sync_copy(gidx_v.at[pl.ds(0, ZROWS)],
                    row_sh.at[pl.ds(sid * ZROWS, ZROWS)])

    tmp16[...] = zeros16i
    for v in range(CHUNK // 16):
        ev = eid_v[pl.ds(v * 16, 16)]
        pcv = plsc.load_gather(tmp16, [ev])
        cnt = tmp16[...]
        pr = zeros16i
        for e in range(E):
            m = ev == jnp.int32(e)
            mi = m.astype(jnp.int32)
            cs = jnp.cumsum(mi)
            pr = jnp.where(m, pcv + cs - 1, pr)
            cnt = cnt + jnp.where(lane16 == jnp.int32(e), jnp.sum(mi), 0)
        tmp16[...] = cnt
        pos_v[pl.ds(v * 16, 16)] = pr

    pltpu.sync_copy(tmp16, cnt_sh.at[sid])
    plsc.subcore_barrier()

    below = zeros16i
    total = zeros16i
    for w2 in range(NTILES):
        pltpu.sync_copy(cnt_sh.at[w2], tmp16)
        cw = tmp16[...]
        below = below + cw * (jnp.int32(w2) < sid).astype(jnp.int32)
        total = total + cw
    padded = ((total + (BM - 1)) >> 7) << 7
    incl = jnp.cumsum(padded)
    mybase = (incl - padded) + below
    base16[...] = mybase

    @pl.when(gwid == 0)
    def _():
        for v3 in range(48 // 16):
            bstart = (lane16 + jnp.int32(v3 * 16)) * jnp.int32(BM)
            be = jnp.zeros((16,), jnp.int32)
            for e in range(E):
                pe = jnp.sum(jnp.where(lane16 == jnp.int32(e), incl, 0))
                be = be + (bstart >= pe).astype(jnp.int32)
            be_v[pl.ds(v3 * 16, 16)] = jnp.minimum(be, jnp.int32(E - 1))
        pltpu.sync_copy(be_v, be_hbm)

    for v in range(CHUNK // 16):
        ev = eid_v[pl.ds(v * 16, 16)]
        bb = plsc.load_gather(base16, [ev])
        ps = bb + pos_v[pl.ds(v * 16, 16)]
        pos_v[pl.ds(v * 16, 16)] = ps
        pidx[v // 8, pl.ds((v % 8) * 16, 16)] = ps
        tidbuf[v // 8, pl.ds((v % 8) * 16, 16)] = (
            jnp.int32(abase + v * 16) + lane16) >> 1
    pltpu.sync_copy(pos_v, pos_hbm.at[pl.ds(abase, CHUNK)])
    for hh in range(CHUNK // 128):
        pltpu.sync_copy(tidbuf.at[hh], row_sh.at[pidx.at[hh]])
    plsc.subcore_barrier()

    pltpu.sync_copy(row_sh.at[pl.ds(gwid * ZROWS, ZROWS)], gidx_v)
    for j in range(ZROWS // 16):
        pltpu.async_copy(x_hbm.at[gidx_v.at[pl.ds(j * 16, 16)]], rowsbuf, sem).wait()
        pltpu.sync_copy(rowsbuf, xs_hbm.at[pl.ds(gwid * ZROWS + j * 16, 16)])


def _sc_route(eflat, x):
    mesh = plsc.VectorSubcoreMesh(core_axis_name="c", subcore_axis_name="s",
                                  num_cores=1)
    f = functools.partial(
        pl.kernel, mesh=mesh,
        compiler_params=pltpu.CompilerParams(needs_layout_passes=False),
        out_type=[
            jax.ShapeDtypeStruct((SPAD, H), jnp.float32),
            jax.ShapeDtypeStruct((48,), jnp.int32),
            jax.ShapeDtypeStruct((A,), jnp.int32),
        ],
        scratch_types=[
            pltpu.VMEM((CHUNK,), jnp.int32),
            pltpu.VMEM((16,), jnp.int32),
            pltpu.VMEM((16,), jnp.int32),
            pltpu.VMEM((CHUNK,), jnp.int32),
            pltpu.VMEM((CHUNK // 128, 128), jnp.int32),
            pltpu.VMEM((CHUNK // 128, 128), jnp.int32),
            pltpu.VMEM((48,), jnp.int32),
            pltpu.VMEM((ZROWS,), jnp.int32),
            pltpu.VMEM((16, H), jnp.float32),
            pltpu.VMEM_SHARED((NTILES, 16), jnp.int32),
            pltpu.VMEM_SHARED((SPAD,), jnp.int32),
            pltpu.SemaphoreType.DMA,
        ],
    )(_route_body)
    return f(eflat, x)


def _ffn_body(be_ref, x_ref, wg_ref, wu_ref, wd_ref, y_ref):
    x = x_ref[...]
    g = jnp.dot(x, wg_ref[0], preferred_element_type=jnp.float32)
    u = jnp.dot(x, wu_ref[0], preferred_element_type=jnp.float32)
    h = g * jax.nn.sigmoid(g) * u
    y_ref[...] = jnp.dot(h, wd_ref[0], preferred_element_type=jnp.float32)


def _ffn(be, xs, W_gate, W_up, W_down):
    grid_spec = pltpu.PrefetchScalarGridSpec(
        num_scalar_prefetch=1,
        grid=(NB,),
        in_specs=[
            pl.BlockSpec((BM, H), lambda i, be: (i, 0)),
            pl.BlockSpec((1, H, I), lambda i, be: (be[i], 0, 0)),
            pl.BlockSpec((1, H, I), lambda i, be: (be[i], 0, 0)),
            pl.BlockSpec((1, I, H), lambda i, be: (be[i], 0, 0)),
        ],
        out_specs=pl.BlockSpec((BM, H), lambda i, be: (i, 0)),
    )
    return pl.pallas_call(
        _ffn_body,
        grid_spec=grid_spec,
        out_shape=jax.ShapeDtypeStruct((SPAD, H), jnp.float32),
    )(be, xs, W_gate, W_up, W_down)


SB = 512


def _combine_body(pe_ref, po_ref, wp_ref, y_ref, out_ref):
    j = pl.program_id(1)
    pe = pe_ref[0, 0, :]
    po = po_ref[0, 0, :]
    wp = wp_ref[...]
    siota = lax.broadcasted_iota(jnp.int32, (pe.shape[0], SB), 1) + j * SB
    P = ((pe[:, None] == siota).astype(jnp.float32) * wp[:, 0:1]
         + (po[:, None] == siota).astype(jnp.float32) * wp[:, 1:2])
    acc = jnp.dot(P, y_ref[...], preferred_element_type=jnp.float32)

    @pl.when(j == 0)
    def _():
        out_ref[...] = acc

    @pl.when(j > 0)
    def _():
        out_ref[...] = out_ref[...] + acc


def _combine(y, pos, w_pad):
    rb = 256
    pr = pos.reshape(T, TOPK)
    pe3 = pr[:, 0].reshape(T // rb, 1, rb)
    po3 = pr[:, 1].reshape(T // rb, 1, rb)
    return pl.pallas_call(
        _combine_body,
        grid=(T // rb, SPAD // SB),
        in_specs=[
            pl.BlockSpec((1, 1, rb), lambda i, j: (i, 0, 0)),
            pl.BlockSpec((1, 1, rb), lambda i, j: (i, 0, 0)),
            pl.BlockSpec((rb, LANES), lambda i, j: (i, 0)),
            pl.BlockSpec((SB, H), lambda i, j: (j, 0)),
        ],
        out_specs=pl.BlockSpec((rb, H), lambda i, j: (i, 0)),
        out_shape=jax.ShapeDtypeStruct((T, H), jnp.float32),
    )(pe3, po3, w_pad, y)


def kernel(hidden_states, gate_W, W_gate, W_up, W_down):
    x = hidden_states.reshape(T, H)
    gw_pad = jnp.zeros((H, LANES), jnp.float32).at[:, :E].set(gate_W)
    logits_pad, ei_pad, w_pad = _router(x, gw_pad)
    router_logits = logits_pad[:, :E]
    eflat = ei_pad[:, :TOPK].reshape(A)

    xs, be, pos = _sc_route(eflat, x)
    y = _ffn(be[:NB], xs, W_gate, W_up, W_down)
    final = _combine(y, pos, w_pad)
    return final.reshape(1, T, H), router_logits
